# baseline (speedup 1.0000x reference)
import jax
import jax.numpy as jnp
from jax import lax
from jax.experimental import pallas as pl
from jax.experimental.pallas import tpu as pltpu
from jax.experimental.pallas import tpu_sc as plsc

N_NODES = 10000
N_EDGES = 320000
D = 128

NC = 2
NS = 16

NG = 16
CPG = D // NG
EH = N_EDGES // 2
N_PAD = 10240

CB = 2048
NFULL = EH // CB
TAIL = EH - NFULL * CB

ACCW = CPG * N_PAD
STGW = CPG * CB
MCW = 10240
NEG_INF = float("-inf")


def _sc_body(values_f, idx_hbm, out_f, scratch_f, acc, stg, dst_buf, sem_v, sem_d):
    cid = lax.axis_index("c")
    sid = lax.axis_index("s")
    g = cid * (NG // NC) + (sid >> 1)
    h = sid & 1
    gg8 = g * CPG
    ebase = h * EH

    lanes = lax.iota(jnp.int32, 16)
    half01 = lanes >> 3
    half10 = 1 - half01
    colpat = lanes & 7
    coladdr = colpat * N_PAD
    vb01 = colpat * CB + half01
    vb10 = colpat * CB + half10

    def init_body(i, carry):
        for c in range(8):
            acc[pl.ds(i * 128 + c * 16, 16)] = jnp.full((16,), NEG_INF, jnp.float32)
        return carry

    lax.fori_loop(0, ACCW // 128, init_body, jnp.int32(0))

    def fire_chunk(k, b):
        off = ebase + k * CB
        for c in range(CPG):
            pltpu.async_copy(
                values_f.at[pl.ds((gg8 + c) * N_EDGES + off, CB)],
                stg.at[pl.ds(b * STGW + c * CB, CB)],
                sem_v,
            )
        pltpu.async_copy(
            idx_hbm.at[pl.ds(off, CB)], dst_buf.at[pl.ds(b * CB, CB)], sem_d
        )

    def wait_chunk():
        for c in range(CPG):
            pltpu.make_async_copy(
                values_f.at[pl.ds(0, CB)], stg.at[pl.ds(0, CB)], sem_v
            ).wait()
        pltpu.make_async_copy(
            idx_hbm.at[pl.ds(0, CB)], dst_buf.at[pl.ds(0, CB)], sem_d
        ).wait()

    fire_chunk(jnp.int32(0), jnp.int32(0))

    def do_pairs(b, npairs):
        dbase = b * CB
        vbase01 = vb01 + (b * STGW)
        vbase10 = vb10 + (b * STGW)

        def pair_body(p, carry):
            for u in range(4):
                pp2 = (p * 4 + u) * 2
                de = plsc.load_gather(dst_buf, [(dbase + pp2) + half01])
                de_sw = plsc.load_gather(dst_buf, [(dbase + pp2) + half10])
                vals = plsc.load_gather(stg, [vbase01 + pp2])
                vals_sw = plsc.load_gather(stg, [vbase10 + pp2])
                deq = de == de_sw
                v_eff = jnp.where(deq, jnp.maximum(vals, vals_sw), vals)
                aaddr = coladdr + de
                cur = plsc.load_gather(acc, [aaddr])
                plsc.store_scatter(acc, [aaddr], jnp.maximum(v_eff, cur))
            return carry

        lax.fori_loop(0, npairs // 4, pair_body, jnp.int32(0))

    def chunk_body(k, carry):
        b = k & 1
        wait_chunk()

        @pl.when(k + 1 < NFULL)
        def _():
            fire_chunk(k + 1, (k + 1) & 1)

        do_pairs(b, CB // 2)
        return carry

    lax.fori_loop(0, NFULL, chunk_body, jnp.int32(0))

    toff = ebase + NFULL * CB
    for c in range(CPG):
        pltpu.sync_copy(
            values_f.at[pl.ds((gg8 + c) * N_EDGES + toff, TAIL)],
            stg.at[pl.ds(c * CB, TAIL)],
        )
    pltpu.sync_copy(idx_hbm.at[pl.ds(toff, TAIL)], dst_buf.at[pl.ds(0, TAIL)])
    def tail_pairs(p, carry):
        for u in range(2):
            pp2 = (p * 2 + u) * 2
            de = plsc.load_gather(dst_buf, [pp2 + half01])
            de_sw = plsc.load_gather(dst_buf, [pp2 + half10])
            vals = plsc.load_gather(stg, [vb01 + pp2])
            vals_sw = plsc.load_gather(stg, [vb10 + pp2])
            deq = de == de_sw
            v_eff = jnp.where(deq, jnp.maximum(vals, vals_sw), vals)
            aaddr = coladdr + de
            cur = plsc.load_gather(acc, [aaddr])
            plsc.store_scatter(acc, [aaddr], jnp.maximum(v_eff, cur))
        return carry

    lax.fori_loop(0, TAIL // 4, tail_pairs, jnp.int32(0))

    sbase = (g * 2 + h) * ACCW
    pbase = (g * 2 + (1 - h)) * ACCW
    pltpu.sync_copy(acc.at[pl.ds(0, ACCW)], scratch_f.at[pl.ds(sbase, ACCW)])
    plsc.subcore_barrier()

    ch0 = h * (ACCW // 2)
    for j in range(ACCW // 2 // MCW):
        foff = ch0 + j * MCW
        pltpu.sync_copy(
            scratch_f.at[pl.ds(pbase + foff, MCW)], stg.at[pl.ds(0, MCW)]
        )

        def merge_body(i, carry):
            a = acc[pl.ds(foff + i * 16, 16)]
            q = stg[pl.ds(i * 16, 16)]
            m = jnp.maximum(a, q)
            m = jnp.where(m == NEG_INF, jnp.float32(0), m)
            stg[pl.ds(i * 16, 16)] = m
            return carry

        lax.fori_loop(0, MCW // 16, merge_body, jnp.int32(0))
        pltpu.sync_copy(
            stg.at[pl.ds(0, MCW)],
            out_f.at[pl.ds(gg8 * N_PAD + foff, MCW)],
        )


def _make_agg():
    mesh = plsc.VectorSubcoreMesh(core_axis_name="c", subcore_axis_name="s")
    return pl.kernel(
        _sc_body,
        out_type=(
            jax.ShapeDtypeStruct((D * N_PAD,), jnp.float32),
            jax.ShapeDtypeStruct((NG * 2 * ACCW,), jnp.float32),
        ),
        mesh=mesh,
        compiler_params=pltpu.CompilerParams(needs_layout_passes=False),
        scratch_types=[
            pltpu.VMEM((ACCW,), jnp.float32),
            pltpu.VMEM((2 * STGW,), jnp.float32),
            pltpu.VMEM((2 * CB,), jnp.int32),
            pltpu.SemaphoreType.DMA,
            pltpu.SemaphoreType.DMA,
        ],
    )


_agg = _make_agg()


def kernel(source_node_representation_with_coefficient, edge_index):
    idx = edge_index[1]
    values_f = source_node_representation_with_coefficient.T.reshape(-1)
    out_f, _ = _agg(values_f, idx)
    return out_f.reshape(D, N_PAD).T[:N_NODES]


# speedup vs baseline: 1.1637x; 1.1637x over previous
import jax
import jax.numpy as jnp
from jax import lax
from jax.experimental import pallas as pl
from jax.experimental.pallas import tpu as pltpu
from jax.experimental.pallas import tpu_sc as plsc

N_NODES = 10000
N_EDGES = 320000
D = 128

NC = 2
NS = 16
NW = NC * NS

NPT = 320
N_PAD = NW * NPT
C = 16000
NCHUNK = N_EDGES // C
G = 128
NEG_INF = float("-inf")


def _sc_body(
    values_hbm, idx_hbm, out_hbm, acc, dst_buf, hit_pack, gid_buf, rows, sem, sem_dst
):
    cid = lax.axis_index("c")
    sid = lax.axis_index("s")
    wid = sid * NC + cid
    lo = wid * NPT

    lanes = lax.iota(jnp.int32, 16)

    def init_body(i, carry):
        for k in range(D // 16):
            acc[i, pl.ds(k * 16, 16)] = jnp.full((16,), NEG_INF, jnp.float32)
        return carry

    lax.fori_loop(0, NPT + 1, init_body, jnp.int32(0))

    pltpu.async_copy(idx_hbm.at[pl.ds(0, C)], dst_buf.at[0], sem_dst)

    def chunk_body(c, carry):
        cb = c & 1
        base = c * C
        pltpu.make_async_copy(
            idx_hbm.at[pl.ds(0, C)], dst_buf.at[0], sem_dst
        ).wait()

        @pl.when(c + 1 < NCHUNK)
        def _():
            pltpu.async_copy(
                idx_hbm.at[pl.ds(base + C, C)], dst_buf.at[(c + 1) & 1], sem_dst
            )

        def filt(q, w):
            jb = q * 64
            packs, cnts = [], []
            for u in range(4):
                d = dst_buf[cb, pl.ds(jb + u * 16, 16)]
                dl = d - lo
                m = (dl >= 0) & (dl < NPT)
                key = jnp.where(m, jnp.int32(0), jnp.int32(1))
                gid = (base + jb + u * 16) + lanes
                pack = (gid << 9) | dl
                _, sp = plsc.sort_key_val(key, pack)
                packs.append(sp)
                cnts.append(plsc.all_reduce_population_count(m))
            c = cnts[0]
            for u in range(1, 4):
                c = jnp.where(lanes == u, cnts[u], c)
            hit_pack[pl.ds(w, 16)] = packs[0]
            w = w + c[0]
            hit_pack[pl.ds(w, 16)] = packs[1]
            w = w + c[1]
            hit_pack[pl.ds(w, 16)] = packs[2]
            w = w + c[2]
            hit_pack[pl.ds(w, 16)] = packs[3]
            return w + c[3]

        w = lax.fori_loop(0, C // 64, filt, jnp.int32(0))

        wpad = ((w + (G - 1)) // G) * G
        trash = jnp.full((16,), NPT, jnp.int32)
        for k in range(G // 16):
            hit_pack[pl.ds(w + k * 16, 16)] = trash

        ngroups = wpad // G

        def unpack_fire(g):
            b = g & 1
            gbase = g * G
            for t in range(G // 16):
                pk = hit_pack[pl.ds(gbase + t * 16, 16)]
                gid_buf[b, pl.ds(t * 16, 16)] = pk >> 9
            pltpu.async_copy(values_hbm.at[gid_buf.at[b]], rows.at[b], sem)

        @pl.when(ngroups > 0)
        def _():
            unpack_fire(jnp.int32(0))

        def group_body(g, carry2):
            b = g & 1
            gbase = g * G
            pltpu.make_async_copy(
                values_hbm.at[gid_buf.at[0]], rows.at[0], sem
            ).wait()

            @pl.when(g + 1 < ngroups)
            def _():
                unpack_fire(g + 1)

            def edge_body(t, carry3):
                pk = hit_pack[pl.ds(gbase + t * 16, 16)]
                dlv = pk & 511
                rbase = t * 16
                for e in range(16):
                    dl = dlv[e]
                    avs = [acc[dl, pl.ds(k * 16, 16)] for k in range(D // 16)]
                    rvs = [
                        rows[b, rbase + e, pl.ds(k * 16, 16)]
                        for k in range(D // 16)
                    ]
                    for k in range(D // 16):
                        acc[dl, pl.ds(k * 16, 16)] = jnp.maximum(avs[k], rvs[k])
                return carry3

            lax.fori_loop(0, G // 16, edge_body, jnp.int32(0))
            return carry2

        lax.fori_loop(0, ngroups, group_body, jnp.int32(0))
        return carry

    lax.fori_loop(0, NCHUNK, chunk_body, jnp.int32(0))

    def fin_body(i, carry):
        for k in range(D // 16):
            v = acc[i, pl.ds(k * 16, 16)]
            acc[i, pl.ds(k * 16, 16)] = jnp.where(v == NEG_INF, jnp.float32(0), v)
        return carry

    lax.fori_loop(0, NPT, fin_body, jnp.int32(0))

    pltpu.sync_copy(acc.at[pl.ds(0, NPT)], out_hbm.at[pl.ds(lo, NPT)])


def _make_agg():
    mesh = plsc.VectorSubcoreMesh(core_axis_name="c", subcore_axis_name="s")
    return pl.kernel(
        _sc_body,
        out_type=jax.ShapeDtypeStruct((N_PAD, D), jnp.float32),
        mesh=mesh,
        compiler_params=pltpu.CompilerParams(needs_layout_passes=False),
        scratch_types=[
            pltpu.VMEM((NPT + 1, D), jnp.float32),
            pltpu.VMEM((2, C), jnp.int32),
            pltpu.VMEM((C + G,), jnp.int32),
            pltpu.VMEM((2, G), jnp.int32),
            pltpu.VMEM((2, G, D), jnp.float32),
            pltpu.SemaphoreType.DMA,
            pltpu.SemaphoreType.DMA,
        ],
    )


_agg = _make_agg()


def kernel(source_node_representation_with_coefficient, edge_index):
    idx = edge_index[1]
    out = _agg(source_node_representation_with_coefficient, idx)
    return out[:N_NODES]
